# baseline (speedup 1.0000x reference)
import functools

import jax
import jax.numpy as jnp
from jax import lax
from jax.experimental import pallas as pl
from jax.experimental.pallas import tpu as pltpu
from jax.experimental.pallas import tpu_sc as plsc

N = 10000
NP = 10240
E = 320000
D = 32
PACK = 128 // D
NPK = NP // PACK
DEG_W = 8

NC = 2
NS = 16
NW = NC * NS
EPT = E // NW
CHUNK = 400
NCHUNK = EPT // CHUNK
RPS = NP // NS


def _sliced_copy(s, src, dst):
    pltpu.sync_copy(src.at[pl.ds(s * RPS, RPS)], dst.at[pl.ds(s * RPS, RPS)])


def _edge_agg_body(y_hbm, ei_hbm, zero_hbm, out_hbm, acc_sh,
                   idx_s, idx_d, rows0, rows1, rows2,
                   sem_i, sem_g0, sem_g1, sem_g2, sem_s0, sem_s1, sem_s2):
    c = lax.axis_index("c")
    s = lax.axis_index("s")
    wid = s * NC + c
    ebase = wid * EPT

    rows = (rows0, rows1, rows2)
    sem_g = (sem_g0, sem_g1, sem_g2)
    sem_s = (sem_s0, sem_s1, sem_s2)

    ih_s = [pltpu.async_copy(
        ei_hbm.at[0, pl.ds(ebase + i * CHUNK, CHUNK)], idx_s.at[i], sem_i)
        for i in range(NCHUNK)]
    ih_d = [pltpu.async_copy(
        ei_hbm.at[1, pl.ds(ebase + i * CHUNK, CHUNK)], idx_d.at[i], sem_i)
        for i in range(NCHUNK)]
    _sliced_copy(s, zero_hbm, acc_sh)
    for h in ih_s:
        h.wait()
    for h in ih_d:
        h.wait()
    plsc.subcore_barrier()

    gh = [None] * NCHUNK
    sh = [None] * NCHUNK
    for i in range(min(2, NCHUNK)):
        gh[i] = pltpu.async_copy(y_hbm.at[idx_s.at[i]], rows[i % 3],
                                 sem_g[i % 3])
    for i in range(NCHUNK):
        b = i % 3
        gh[i].wait()
        sh[i] = pltpu.async_copy(rows[b], acc_sh.at[idx_d.at[i]], sem_s[b],
                                 add=True)
        if i >= 1:
            sh[i - 1].wait()
        j = i + 2
        if j < NCHUNK:
            gh[j] = pltpu.async_copy(y_hbm.at[idx_s.at[j]], rows[j % 3],
                                     sem_g[j % 3])
    sh[NCHUNK - 1].wait()

    plsc.subcore_barrier()
    _sliced_copy(s, acc_sh, out_hbm.at[c])


@functools.cache
def _get_edge_agg():
    mesh = plsc.VectorSubcoreMesh(core_axis_name="c", subcore_axis_name="s",
                                  num_cores=NC, num_subcores=NS)
    return pl.kernel(
        _edge_agg_body,
        out_type=jax.ShapeDtypeStruct((NC, NP, D), jnp.float32),
        mesh=mesh,
        compiler_params=pltpu.CompilerParams(use_tc_tiling_on_sc=False),
        scratch_types=[
            pltpu.VMEM_SHARED((NP, D), jnp.float32),
            pltpu.VMEM((NCHUNK, CHUNK), jnp.int32),
            pltpu.VMEM((NCHUNK, CHUNK), jnp.int32),
            pltpu.VMEM((CHUNK, D), jnp.float32),
            pltpu.VMEM((CHUNK, D), jnp.float32),
            pltpu.VMEM((CHUNK, D), jnp.float32),
            pltpu.SemaphoreType.DMA,
            pltpu.SemaphoreType.DMA,
            pltpu.SemaphoreType.DMA,
            pltpu.SemaphoreType.DMA,
            pltpu.SemaphoreType.DMA,
            pltpu.SemaphoreType.DMA,
            pltpu.SemaphoreType.DMA,
        ],
    )


def _deg_body(ei_hbm, ones_hbm, zero_hbm, out_hbm,
              deg_sh, idx_d, ones_v, sem_i, sem_s):
    c = lax.axis_index("c")
    s = lax.axis_index("s")
    wid = s * NC + c
    ebase = wid * EPT

    ih = [pltpu.async_copy(
        ei_hbm.at[1, pl.ds(ebase + i * CHUNK, CHUNK)], idx_d.at[i], sem_i)
        for i in range(NCHUNK)]
    oh = pltpu.async_copy(ones_hbm, ones_v, sem_i)
    _sliced_copy(s, zero_hbm, deg_sh)
    for h in ih:
        h.wait()
    oh.wait()
    plsc.subcore_barrier()

    sh = [pltpu.async_copy(ones_v, deg_sh.at[idx_d.at[i]], sem_s, add=True)
          for i in range(NCHUNK)]
    for h in sh:
        h.wait()
    plsc.subcore_barrier()
    _sliced_copy(s, deg_sh, out_hbm.at[c])


@functools.cache
def _get_deg():
    mesh = plsc.VectorSubcoreMesh(core_axis_name="c", subcore_axis_name="s",
                                  num_cores=NC, num_subcores=NS)
    return pl.kernel(
        _deg_body,
        out_type=jax.ShapeDtypeStruct((NC, NP, DEG_W), jnp.float32),
        mesh=mesh,
        compiler_params=pltpu.CompilerParams(use_tc_tiling_on_sc=False),
        scratch_types=[
            pltpu.VMEM_SHARED((NP, DEG_W), jnp.float32),
            pltpu.VMEM((NCHUNK, CHUNK), jnp.int32),
            pltpu.VMEM((CHUNK, DEG_W), jnp.float32),
            pltpu.SemaphoreType.DMA,
            pltpu.SemaphoreType.DMA,
        ],
    )


def _tc0_body(x4_ref, w_ref, y_ref):
    y_ref[...] = jnp.dot(x4_ref[...], w_ref[...],
                         preferred_element_type=jnp.float32)


def _tc0(x4, w0s):
    return pl.pallas_call(
        _tc0_body,
        out_shape=jax.ShapeDtypeStruct((NPK, 128), jnp.float32),
    )(x4, w0s)


def _tcdinv_body(degp_ref, s_ref, o_ref):
    r8 = lax.rsqrt(degp_ref[0] + degp_ref[1] + 1.0)
    o_ref[...] = jnp.dot(r8, s_ref[...], preferred_element_type=jnp.float32,
                         precision=lax.Precision.HIGHEST)


def _tcdinv(degp_pk, s_mat):
    return pl.pallas_call(
        _tcdinv_body,
        out_shape=jax.ShapeDtypeStruct((NP // 16, 512), jnp.float32),
    )(degp_pk, s_mat)


def _tc_mid_body(y_ref, accp_ref, dinv_ref, b_ref, w_ref, o_ref):
    dinv = dinv_ref[...]
    h = (y_ref[...] + accp_ref[0] + accp_ref[1]) * dinv + b_ref[...]
    z = jnp.where(h >= 0.0, h, 0.01 * h)
    o_ref[...] = jnp.dot(z, w_ref[...],
                         preferred_element_type=jnp.float32) * dinv


def _tc_mid(y_pk, accp_pk, dinv_pk, b128, wbd):
    return pl.pallas_call(
        _tc_mid_body,
        out_shape=jax.ShapeDtypeStruct((NPK, 128), jnp.float32),
    )(y_pk, accp_pk, dinv_pk, b128, wbd)


def kernel(x, edge_index, edge_attr, W0, b0, W1, b1, W2, b2):
    zero_nd = jnp.zeros((NP, D), jnp.float32)
    zero_nw = jnp.zeros((NP, DEG_W), jnp.float32)
    ones_cw = jnp.ones((CHUNK, DEG_W), jnp.float32)
    eye4 = jnp.eye(PACK, dtype=jnp.float32)
    w0s = jnp.kron(eye4, W0)
    w1bd = jnp.kron(eye4, W1)
    w2bd = jnp.kron(eye4, W2)
    b0p = jnp.tile(b0, PACK).reshape(1, 128)
    b1p = jnp.tile(b1, PACK).reshape(1, 128)
    b2p = jnp.tile(b2, PACK).reshape(1, 128)
    x4 = jnp.pad(x, ((0, NP - N), (0, 0))).reshape(NPK, PACK * 128)

    deg_fn = _get_deg()
    agg_fn = _get_edge_agg()
    sel = jnp.zeros((DEG_W, D), jnp.float32).at[0].set(1.0)
    s_mat = jnp.kron(jnp.eye(16, dtype=jnp.float32), sel)
    degp = deg_fn(edge_index, ones_cw, zero_nw)
    xw_pk = _tc0(x4, w0s)
    dinv_pk = _tcdinv(degp.reshape(NC, NP // 16, 128), s_mat).reshape(NPK, 128)
    y0_pk = xw_pk * dinv_pk
    acc0 = agg_fn(y0_pk.reshape(NP, D), edge_index, zero_nd)
    y1_pk = _tc_mid(y0_pk, acc0.reshape(NC, NPK, 128), dinv_pk, b0p, w1bd)
    acc1 = agg_fn(y1_pk.reshape(NP, D), edge_index, zero_nd)
    y2_pk = _tc_mid(y1_pk, acc1.reshape(NC, NPK, 128), dinv_pk, b1p, w2bd)
    acc2 = agg_fn(y2_pk.reshape(NP, D), edge_index, zero_nd)
    acc2_pk = acc2.reshape(NC, NPK, 128)
    out_pk = (y2_pk + acc2_pk[0] + acc2_pk[1]) * dinv_pk + b2p
    return out_pk[:N * D // 128].reshape(N, D)


# speedup vs baseline: 1.1003x; 1.1003x over previous
import functools

import jax
import jax.numpy as jnp
from jax import lax
from jax.experimental import pallas as pl
from jax.experimental.pallas import tpu as pltpu
from jax.experimental.pallas import tpu_sc as plsc

N = 10000
NP = 10240
E = 320000
D = 32
PACK = 128 // D
NPK = NP // PACK
DEG_W = 8

NC = 2
NS = 16
NW = NC * NS
EPT = E // NW
CHUNK = 400
NCHUNK = EPT // CHUNK
RPS = NP // NS


def _sliced_copy(s, src, dst):
    pltpu.sync_copy(src.at[pl.ds(s * RPS, RPS)], dst.at[pl.ds(s * RPS, RPS)])


def _edge_agg_body(y_hbm, ei_hbm, zero_hbm, out_hbm, acc_sh,
                   idx_s, idx_d, rows0, rows1, rows2,
                   sem_i, sem_g0, sem_g1, sem_g2, sem_s0, sem_s1, sem_s2):
    c = lax.axis_index("c")
    s = lax.axis_index("s")
    wid = s * NC + c
    ebase = wid * EPT

    rows = (rows0, rows1, rows2)
    sem_g = (sem_g0, sem_g1, sem_g2)
    sem_s = (sem_s0, sem_s1, sem_s2)

    ih_s = [pltpu.async_copy(
        ei_hbm.at[0, pl.ds(ebase + i * CHUNK, CHUNK)], idx_s.at[i], sem_i)
        for i in range(NCHUNK)]
    ih_d = [pltpu.async_copy(
        ei_hbm.at[1, pl.ds(ebase + i * CHUNK, CHUNK)], idx_d.at[i], sem_i)
        for i in range(NCHUNK)]
    _sliced_copy(s, zero_hbm, acc_sh)
    for h in ih_s:
        h.wait()
    for h in ih_d:
        h.wait()
    plsc.subcore_barrier()

    gh = [None] * NCHUNK
    sh = [None] * NCHUNK
    for i in range(min(2, NCHUNK)):
        gh[i] = pltpu.async_copy(y_hbm.at[idx_s.at[i]], rows[i % 3],
                                 sem_g[i % 3])
    for i in range(NCHUNK):
        b = i % 3
        gh[i].wait()
        sh[i] = pltpu.async_copy(rows[b], acc_sh.at[idx_d.at[i]], sem_s[b],
                                 add=True)
        if i >= 1:
            sh[i - 1].wait()
        j = i + 2
        if j < NCHUNK:
            gh[j] = pltpu.async_copy(y_hbm.at[idx_s.at[j]], rows[j % 3],
                                     sem_g[j % 3])
    sh[NCHUNK - 1].wait()

    plsc.subcore_barrier()
    _sliced_copy(s, acc_sh, out_hbm.at[c])


@functools.cache
def _get_edge_agg():
    mesh = plsc.VectorSubcoreMesh(core_axis_name="c", subcore_axis_name="s",
                                  num_cores=NC, num_subcores=NS)
    return pl.kernel(
        _edge_agg_body,
        out_type=jax.ShapeDtypeStruct((NC, NP, D), jnp.float32),
        mesh=mesh,
        compiler_params=pltpu.CompilerParams(use_tc_tiling_on_sc=False),
        scratch_types=[
            pltpu.VMEM_SHARED((NP, D), jnp.float32),
            pltpu.VMEM((NCHUNK, CHUNK), jnp.int32),
            pltpu.VMEM((NCHUNK, CHUNK), jnp.int32),
            pltpu.VMEM((CHUNK, D), jnp.float32),
            pltpu.VMEM((CHUNK, D), jnp.float32),
            pltpu.VMEM((CHUNK, D), jnp.float32),
            pltpu.SemaphoreType.DMA,
            pltpu.SemaphoreType.DMA,
            pltpu.SemaphoreType.DMA,
            pltpu.SemaphoreType.DMA,
            pltpu.SemaphoreType.DMA,
            pltpu.SemaphoreType.DMA,
            pltpu.SemaphoreType.DMA,
        ],
    )


def _deg_body(ei_hbm, ones_hbm, zero_hbm, out_hbm,
              deg_sh, idx_d, ones_v, sem_i, sem_s):
    c = lax.axis_index("c")
    s = lax.axis_index("s")
    wid = s * NC + c
    ebase = wid * EPT

    ih = [pltpu.async_copy(
        ei_hbm.at[1, pl.ds(ebase + i * CHUNK, CHUNK)], idx_d.at[i], sem_i)
        for i in range(NCHUNK)]
    oh = pltpu.async_copy(ones_hbm, ones_v, sem_i)
    _sliced_copy(s, zero_hbm, deg_sh)
    for h in ih:
        h.wait()
    oh.wait()
    plsc.subcore_barrier()

    sh = [pltpu.async_copy(ones_v, deg_sh.at[idx_d.at[i]], sem_s, add=True)
          for i in range(NCHUNK)]
    for h in sh:
        h.wait()
    plsc.subcore_barrier()
    _sliced_copy(s, deg_sh, out_hbm.at[c])


@functools.cache
def _get_deg():
    mesh = plsc.VectorSubcoreMesh(core_axis_name="c", subcore_axis_name="s",
                                  num_cores=NC, num_subcores=NS)
    return pl.kernel(
        _deg_body,
        out_type=jax.ShapeDtypeStruct((NC, NP, DEG_W), jnp.float32),
        mesh=mesh,
        compiler_params=pltpu.CompilerParams(use_tc_tiling_on_sc=False),
        scratch_types=[
            pltpu.VMEM_SHARED((NP, DEG_W), jnp.float32),
            pltpu.VMEM((NCHUNK, CHUNK), jnp.int32),
            pltpu.VMEM((CHUNK, DEG_W), jnp.float32),
            pltpu.SemaphoreType.DMA,
            pltpu.SemaphoreType.DMA,
        ],
    )


def _tc0_body(x4_ref, w_ref, y_ref):
    y_ref[...] = jnp.dot(x4_ref[...], w_ref[...],
                         preferred_element_type=jnp.float32)


def _tc0(x4, w0s):
    return pl.pallas_call(
        _tc0_body,
        out_shape=jax.ShapeDtypeStruct((NPK, 128), jnp.float32),
    )(x4, w0s)


def _tcdinv_body(degp_ref, s_ref, o_ref):
    r8 = lax.rsqrt(degp_ref[0] + degp_ref[1] + 1.0)
    o_ref[...] = jnp.dot(r8, s_ref[...], preferred_element_type=jnp.float32,
                         precision=lax.Precision.HIGHEST)


def _tc_fin_body(y_ref, accp_ref, dinv_ref, b_ref, o_ref):
    o = (y_ref[...] + accp_ref[0] + accp_ref[1]) * dinv_ref[...] + b_ref[...]
    o_ref[...] = o[:N * D // 128]


def _tc_fin(y_pk, accp_pk, dinv_pk, b128):
    return pl.pallas_call(
        _tc_fin_body,
        out_shape=jax.ShapeDtypeStruct((N * D // 128, 128), jnp.float32),
    )(y_pk, accp_pk, dinv_pk, b128)


def _tcdinv(degp_pk, s_mat):
    return pl.pallas_call(
        _tcdinv_body,
        out_shape=jax.ShapeDtypeStruct((NP // 16, 512), jnp.float32),
    )(degp_pk, s_mat)


def _tc_mid_body(y_ref, accp_ref, dinv_ref, b_ref, w_ref, o_ref):
    dinv = dinv_ref[...]
    h = (y_ref[...] + accp_ref[0] + accp_ref[1]) * dinv + b_ref[...]
    z = jnp.where(h >= 0.0, h, 0.01 * h)
    o_ref[...] = jnp.dot(z, w_ref[...],
                         preferred_element_type=jnp.float32) * dinv


def _tc_mid(y_pk, accp_pk, dinv_pk, b128, wbd):
    return pl.pallas_call(
        _tc_mid_body,
        out_shape=jax.ShapeDtypeStruct((NPK, 128), jnp.float32),
    )(y_pk, accp_pk, dinv_pk, b128, wbd)


def kernel(x, edge_index, edge_attr, W0, b0, W1, b1, W2, b2):
    zero_nd = jnp.zeros((NP, D), jnp.float32)
    zero_nw = jnp.zeros((NP, DEG_W), jnp.float32)
    ones_cw = jnp.ones((CHUNK, DEG_W), jnp.float32)
    eye4 = jnp.eye(PACK, dtype=jnp.float32)
    w0s = jnp.kron(eye4, W0)
    w1bd = jnp.kron(eye4, W1)
    w2bd = jnp.kron(eye4, W2)
    b0p = jnp.tile(b0, PACK).reshape(1, 128)
    b1p = jnp.tile(b1, PACK).reshape(1, 128)
    b2p = jnp.tile(b2, PACK).reshape(1, 128)
    x4 = jnp.pad(x, ((0, NP - N), (0, 0))).reshape(NPK, PACK * 128)

    deg_fn = _get_deg()
    agg_fn = _get_edge_agg()
    sel = jnp.zeros((DEG_W, D), jnp.float32).at[0].set(1.0)
    s_mat = jnp.kron(jnp.eye(16, dtype=jnp.float32), sel)
    degp = deg_fn(edge_index, ones_cw, zero_nw)
    xw_pk = _tc0(x4, w0s)
    dinv_pk = _tcdinv(degp.reshape(NC, NP // 16, 128), s_mat).reshape(NPK, 128)
    y0_pk = xw_pk * dinv_pk
    acc0 = agg_fn(y0_pk.reshape(NP, D), edge_index, zero_nd)
    y1_pk = _tc_mid(y0_pk, acc0.reshape(NC, NPK, 128), dinv_pk, b0p, w1bd)
    acc1 = agg_fn(y1_pk.reshape(NP, D), edge_index, zero_nd)
    y2_pk = _tc_mid(y1_pk, acc1.reshape(NC, NPK, 128), dinv_pk, b1p, w2bd)
    acc2 = agg_fn(y2_pk.reshape(NP, D), edge_index, zero_nd)
    out_pk = _tc_fin(y2_pk, acc2.reshape(NC, NPK, 128), dinv_pk, b2p)
    return out_pk.reshape(N, D)


# speedup vs baseline: 1.1486x; 1.0439x over previous
import functools

import jax
import jax.numpy as jnp
from jax import lax
from jax.experimental import pallas as pl
from jax.experimental.pallas import tpu as pltpu
from jax.experimental.pallas import tpu_sc as plsc

N = 10000
NP = 10240
E = 320000
D = 32
PACK = 8
LW = D * PACK
NPK = NP // PACK
DEG_W = 32

NC = 2
NS = 16
NW = NC * NS
EPT = E // NW
CHUNK = 1000
NCHUNK = EPT // CHUNK
RPS = NP // NS

F32 = jnp.float32
BF16 = jnp.bfloat16


def _sliced_copy(s, src, dst):
    pltpu.sync_copy(src.at[pl.ds(s * RPS, RPS)], dst.at[pl.ds(s * RPS, RPS)])


def _edge_agg_body(y_hbm, ei_hbm, zero_hbm, out_hbm, acc_sh,
                   idx_s, idx_d, rows0, rows1, rows2,
                   sem_i, sem_g0, sem_g1, sem_g2, sem_s0, sem_s1, sem_s2):
    c = lax.axis_index("c")
    s = lax.axis_index("s")
    wid = s * NC + c
    ebase = wid * EPT

    rows = (rows0, rows1, rows2)
    sem_g = (sem_g0, sem_g1, sem_g2)
    sem_s = (sem_s0, sem_s1, sem_s2)

    ih_s = [pltpu.async_copy(
        ei_hbm.at[0, pl.ds(ebase + i * CHUNK, CHUNK)], idx_s.at[i], sem_i)
        for i in range(NCHUNK)]
    ih_d = [pltpu.async_copy(
        ei_hbm.at[1, pl.ds(ebase + i * CHUNK, CHUNK)], idx_d.at[i], sem_i)
        for i in range(NCHUNK)]
    _sliced_copy(s, zero_hbm, acc_sh)
    for h in ih_s:
        h.wait()
    for h in ih_d:
        h.wait()
    plsc.subcore_barrier()

    gh = [None] * NCHUNK
    sh = [None] * NCHUNK
    for i in range(min(2, NCHUNK)):
        gh[i] = pltpu.async_copy(y_hbm.at[idx_s.at[i]], rows[i % 3],
                                 sem_g[i % 3])
    for i in range(NCHUNK):
        b = i % 3
        gh[i].wait()
        sh[i] = pltpu.async_copy(rows[b], acc_sh.at[idx_d.at[i]], sem_s[b],
                                 add=True)
        if i >= 1:
            sh[i - 1].wait()
        j = i + 2
        if j < NCHUNK:
            gh[j] = pltpu.async_copy(y_hbm.at[idx_s.at[j]], rows[j % 3],
                                     sem_g[j % 3])
    sh[NCHUNK - 1].wait()

    plsc.subcore_barrier()
    _sliced_copy(s, acc_sh, out_hbm.at[c])


@functools.cache
def _get_edge_agg():
    mesh = plsc.VectorSubcoreMesh(core_axis_name="c", subcore_axis_name="s",
                                  num_cores=NC, num_subcores=NS)
    return pl.kernel(
        _edge_agg_body,
        out_type=jax.ShapeDtypeStruct((NC, NP, D), BF16),
        mesh=mesh,
        compiler_params=pltpu.CompilerParams(use_tc_tiling_on_sc=False),
        scratch_types=[
            pltpu.VMEM_SHARED((NP, D), BF16),
            pltpu.VMEM((NCHUNK, CHUNK), jnp.int32),
            pltpu.VMEM((NCHUNK, CHUNK), jnp.int32),
            pltpu.VMEM((CHUNK, D), BF16),
            pltpu.VMEM((CHUNK, D), BF16),
            pltpu.VMEM((CHUNK, D), BF16),
            pltpu.SemaphoreType.DMA,
            pltpu.SemaphoreType.DMA,
            pltpu.SemaphoreType.DMA,
            pltpu.SemaphoreType.DMA,
            pltpu.SemaphoreType.DMA,
            pltpu.SemaphoreType.DMA,
            pltpu.SemaphoreType.DMA,
        ],
    )


def _deg_body(ei_hbm, ones_hbm, zero_hbm, out_hbm,
              deg_sh, idx_d, ones_v, sem_i, sem_s):
    c = lax.axis_index("c")
    s = lax.axis_index("s")
    wid = s * NC + c
    ebase = wid * EPT

    ih = [pltpu.async_copy(
        ei_hbm.at[1, pl.ds(ebase + i * CHUNK, CHUNK)], idx_d.at[i], sem_i)
        for i in range(NCHUNK)]
    oh = pltpu.async_copy(ones_hbm, ones_v, sem_i)
    _sliced_copy(s, zero_hbm, deg_sh)
    for h in ih:
        h.wait()
    oh.wait()
    plsc.subcore_barrier()

    sh = [pltpu.async_copy(ones_v, deg_sh.at[idx_d.at[i]], sem_s, add=True)
          for i in range(NCHUNK)]
    for h in sh:
        h.wait()
    plsc.subcore_barrier()
    _sliced_copy(s, deg_sh, out_hbm.at[c])


@functools.cache
def _get_deg():
    mesh = plsc.VectorSubcoreMesh(core_axis_name="c", subcore_axis_name="s",
                                  num_cores=NC, num_subcores=NS)
    return pl.kernel(
        _deg_body,
        out_type=jax.ShapeDtypeStruct((NC, NP, DEG_W), BF16),
        mesh=mesh,
        compiler_params=pltpu.CompilerParams(use_tc_tiling_on_sc=False),
        scratch_types=[
            pltpu.VMEM_SHARED((NP, DEG_W), BF16),
            pltpu.VMEM((NCHUNK, CHUNK), jnp.int32),
            pltpu.VMEM((CHUNK, DEG_W), BF16),
            pltpu.SemaphoreType.DMA,
            pltpu.SemaphoreType.DMA,
        ],
    )


def _tc0_body(x8_ref, w_ref, y_ref):
    y_ref[...] = jnp.dot(x8_ref[...], w_ref[...], preferred_element_type=F32)


def _tc0(x8, w0s):
    return pl.pallas_call(
        _tc0_body,
        out_shape=jax.ShapeDtypeStruct((NPK, LW), F32),
    )(x8, w0s)


def _tcdinv_body(degp_ref, o_ref):
    d = degp_ref[0].astype(F32) + degp_ref[1].astype(F32) + 1.0
    o_ref[...] = lax.rsqrt(d)


def _tcdinv(degp_pk):
    return pl.pallas_call(
        _tcdinv_body,
        out_shape=jax.ShapeDtypeStruct((NPK, LW), F32),
    )(degp_pk)


def _tc_mid_body(y_ref, accp_ref, dinv_ref, b_ref, w_ref, o_ref):
    dinv = dinv_ref[...]
    a = (y_ref[...].astype(F32) + accp_ref[0].astype(F32)
         + accp_ref[1].astype(F32))
    h = a * dinv + b_ref[...]
    z = jnp.where(h >= 0.0, h, 0.01 * h)
    o_ref[...] = (jnp.dot(z, w_ref[...],
                          preferred_element_type=F32) * dinv).astype(BF16)


def _tc_mid(y_pk, accp_pk, dinv_pk, b256, wbd):
    return pl.pallas_call(
        _tc_mid_body,
        out_shape=jax.ShapeDtypeStruct((NPK, LW), BF16),
    )(y_pk, accp_pk, dinv_pk, b256, wbd)


def _tc_fin_body(y_ref, accp_ref, dinv_ref, b_ref, o_ref):
    a = (y_ref[...].astype(F32) + accp_ref[0].astype(F32)
         + accp_ref[1].astype(F32))
    o = a * dinv_ref[...] + b_ref[...]
    o_ref[...] = o[:N * D // LW]


def _tc_fin(y_pk, accp_pk, dinv_pk, b256):
    return pl.pallas_call(
        _tc_fin_body,
        out_shape=jax.ShapeDtypeStruct((N * D // LW, LW), F32),
    )(y_pk, accp_pk, dinv_pk, b256)


def kernel(x, edge_index, edge_attr, W0, b0, W1, b1, W2, b2):
    zero_nd = jnp.zeros((NP, D), BF16)
    zero_nw = jnp.zeros((NP, DEG_W), BF16)
    ones_cw = jnp.ones((CHUNK, DEG_W), BF16)
    eye8 = jnp.eye(PACK, dtype=F32)
    w0s = jnp.kron(eye8, W0)
    w1bd = jnp.kron(eye8, W1)
    w2bd = jnp.kron(eye8, W2)
    b0p = jnp.tile(b0, PACK).reshape(1, LW)
    b1p = jnp.tile(b1, PACK).reshape(1, LW)
    b2p = jnp.tile(b2, PACK).reshape(1, LW)
    x8 = jnp.pad(x, ((0, NP - N), (0, 0))).reshape(NPK, PACK * 128)

    deg_fn = _get_deg()
    agg_fn = _get_edge_agg()
    degp = deg_fn(edge_index, ones_cw, zero_nw)
    xw_pk = _tc0(x8, w0s)
    dinv_pk = _tcdinv(degp.reshape(NC, NPK, LW))
    y0_pk = (xw_pk * dinv_pk).astype(BF16)
    acc0 = agg_fn(y0_pk.reshape(NP, D), edge_index, zero_nd)
    y1_pk = _tc_mid(y0_pk, acc0.reshape(NC, NPK, LW), dinv_pk, b0p, w1bd)
    acc1 = agg_fn(y1_pk.reshape(NP, D), edge_index, zero_nd)
    y2_pk = _tc_mid(y1_pk, acc1.reshape(NC, NPK, LW), dinv_pk, b1p, w2bd)
    acc2 = agg_fn(y2_pk.reshape(NP, D), edge_index, zero_nd)
    out_pk = _tc_fin(y2_pk, acc2.reshape(NC, NPK, LW), dinv_pk, b2p)
    return out_pk.reshape(N, D)
